# baseline (speedup 1.0000x reference)
import jax
import jax.numpy as jnp
from jax.experimental import pallas as pl
from jax.experimental.pallas import tpu as pltpu

_NS = 2
_NT = 2


def _netvlad_kernel(*refs):
    x_refs = refs[:_NS]
    w_ref, c_ref, o_ref = refs[_NS:_NS + 3]
    ax_ref, asum_ref = refs[_NS + 3:]
    K, C = w_ref.shape
    Cs = C // _NS
    t = pl.program_id(1)
    x_bf = [x_refs[j][0].astype(jnp.bfloat16) for j in range(_NS)]
    w_bf = w_ref[...].astype(jnp.bfloat16)
    logits = jnp.dot(w_bf[:, 0:Cs], x_bf[0],
                     preferred_element_type=jnp.float32)
    for j in range(1, _NS):
        logits = logits + jnp.dot(w_bf[:, j * Cs:(j + 1) * Cs],
                                  x_bf[j],
                                  preferred_element_type=jnp.float32)
    m = jnp.max(logits, axis=0, keepdims=True)
    e = jnp.exp(logits - m)
    s = jnp.sum(e, axis=0, keepdims=True)
    a = e / s
    a_bf = a.astype(jnp.bfloat16)
    asum_t = jnp.sum(a, axis=1, keepdims=True)
    ax_t = [
        jax.lax.dot_general(
            a_bf, x_bf[j], (((1,), (1,)), ((), ())),
            preferred_element_type=jnp.float32)
        for j in range(_NS)
    ]
    ax_t_full = jnp.concatenate(ax_t, axis=1)

    @pl.when(t == 0)
    def _():
        ax_ref[...] = ax_t_full
        asum_ref[...] = jnp.broadcast_to(asum_t, asum_ref.shape)

    @pl.when(t > 0)
    def _():
        ax_ref[...] += ax_t_full
        asum_ref[...] += jnp.broadcast_to(asum_t, asum_ref.shape)

    @pl.when(t == _NT - 1)
    def _():
        vlad = ax_ref[...] - asum_ref[:, 0:1] * c_ref[...]
        sq = jnp.sum(vlad * vlad)
        inv = 1.0 / jnp.maximum(jnp.sqrt(sq), 1e-12)
        o_ref[0] = vlad * inv


def kernel(x, conv_w, centroids):
    B, C, N = x.shape
    K = conv_w.shape[0]
    Cs = C // _NS
    Nc = N // _NT
    x_specs = [
        pl.BlockSpec((1, Cs, Nc), lambda b, t, j=j: (b, j, t))
        for j in range(_NS)
    ]
    out = pl.pallas_call(
        _netvlad_kernel,
        grid=(B, _NT),
        in_specs=x_specs + [
            pl.BlockSpec((K, C), lambda b, t: (0, 0)),
            pl.BlockSpec((K, C), lambda b, t: (0, 0)),
        ],
        out_specs=pl.BlockSpec((1, K, C), lambda b, t: (b, 0, 0)),
        out_shape=jax.ShapeDtypeStruct((B, K, C), jnp.float32),
        scratch_shapes=[
            pltpu.VMEM((K, C), jnp.float32),
            pltpu.VMEM((K, 128), jnp.float32),
        ],
        compiler_params=pltpu.CompilerParams(
            dimension_semantics=("arbitrary", "arbitrary"),
        ),
    )(*([x] * _NS), conv_w, centroids)
    return out.reshape(B, K * C)


# speedup vs baseline: 1.5556x; 1.5556x over previous
import jax
import jax.numpy as jnp
from jax.experimental import pallas as pl
from jax.experimental.pallas import tpu as pltpu


def _netvlad_kernel(xa_ref, xb_ref, w_ref, c_ref, o_ref):
    K, C = w_ref.shape
    w_bf = w_ref[...].astype(jnp.bfloat16)
    c = c_ref[...]
    for h, x_ref in enumerate((xa_ref, xb_ref)):
        x_bf = x_ref[0, 0].astype(jnp.bfloat16)
        logits = jnp.dot(w_bf, x_bf, preferred_element_type=jnp.float32)
        m = jnp.max(logits, axis=0, keepdims=True)
        e = jnp.exp(logits - m)
        s = jnp.sum(e, axis=0, keepdims=True)
        a = e / s
        a_sum = jnp.sum(a, axis=1, keepdims=True)
        a_bf = a.astype(jnp.bfloat16)
        ax = jax.lax.dot_general(
            a_bf, x_bf, (((1,), (1,)), ((), ())),
            preferred_element_type=jnp.float32)
        vlad = ax - a_sum * c
        sq = jnp.sum(vlad * vlad)
        inv = 1.0 / jnp.maximum(jnp.sqrt(sq), 1e-12)
        o_ref[h, 0] = vlad * inv


def kernel(x, conv_w, centroids):
    B, C, N = x.shape
    K = conv_w.shape[0]
    H = B // 2
    x4 = x.reshape(2, H, C, N)
    out = pl.pallas_call(
        _netvlad_kernel,
        grid=(H,),
        in_specs=[
            pl.BlockSpec((1, 1, C, N), lambda b: (0, b, 0, 0)),
            pl.BlockSpec((1, 1, C, N), lambda b: (1, b, 0, 0)),
            pl.BlockSpec((K, C), lambda b: (0, 0)),
            pl.BlockSpec((K, C), lambda b: (0, 0)),
        ],
        out_specs=pl.BlockSpec((2, 1, K, C), lambda b: (0, b, 0, 0)),
        out_shape=jax.ShapeDtypeStruct((2, H, K, C), jnp.float32),
        compiler_params=pltpu.CompilerParams(
            dimension_semantics=("arbitrary",),
        ),
    )(x4, x4, conv_w, centroids)
    return out.reshape(B, K * C)
